# baseline (speedup 1.0000x reference)
import functools
import math

import jax
import jax.numpy as jnp
from jax import lax
from jax.experimental import pallas as pl
from jax.experimental.pallas import tpu as pltpu
from jax.experimental.pallas import tpu_sc as plsc

VOCAB = 100000
DIM = 64
SEQ = 1500
BATCH = 1024
SEQ_PAD = 2048
SEQ_BLOCKS = SEQ_PAD // 128
N_CHUNK = 12
TAIL = SEQ - 11 * 128

NC = 2
NS = 16
NW = NC * NS
ROWS_PER_W = BATCH // NW

NBUF = 2


def _pe_table():
    position = jnp.arange(SEQ, dtype=jnp.float32)[:, None]
    div_term = jnp.exp(
        jnp.arange(0, DIM, 2, dtype=jnp.float32) * (-(math.log(10000.0) / DIM))
    )
    ang = position * div_term
    pe = jnp.zeros((SEQ, DIM), dtype=jnp.float32)
    pe = pe.at[:, 0::2].set(jnp.sin(ang))
    pe = pe.at[:, 1::2].set(jnp.cos(ang))
    return jnp.pad(pe, ((0, N_CHUNK * 128 - SEQ), (0, 0)))


@functools.partial(
    pl.kernel,
    mesh=plsc.VectorSubcoreMesh(core_axis_name="c", subcore_axis_name="s"),
    out_type=jax.ShapeDtypeStruct((BATCH, SEQ, DIM), jnp.float32),
    scratch_types=(
        [pltpu.VMEM((128, DIM), jnp.float32)]
        + [pltpu.VMEM((8, 128), jnp.int32) for _ in range(NBUF)]
        + [pltpu.VMEM((128, 128), jnp.float32) for _ in range(NBUF)]
        + [pltpu.VMEM((128, DIM), jnp.float32) for _ in range(NBUF)]
        + [pltpu.VMEM((TAIL, DIM), jnp.float32) for _ in range(NBUF)]
        + [pltpu.SemaphoreType.DMA for _ in range(3 * NBUF)]
    ),
)
def _seq_encode(tok_hbm, pe_hbm, table_hbm, out_hbm, pe_v, *scratch):
    idx = scratch[:NBUF]
    rows = scratch[NBUF:2 * NBUF]
    outv = scratch[2 * NBUF:3 * NBUF]
    tailv = scratch[3 * NBUF:4 * NBUF]
    sem_i = scratch[4 * NBUF:5 * NBUF]
    sem_g = scratch[5 * NBUF:6 * NBUF]
    sem_s = scratch[6 * NBUF:7 * NBUF]
    wid = lax.axis_index("s") * NC + lax.axis_index("c")

    for k in range(N_CHUNK):
        is_tail = k == N_CHUNK - 1
        sb = (k // 8) * 8
        kb = k % 8
        cs = TAIL if is_tail else 128

        pltpu.sync_copy(pe_hbm.at[pl.ds(128 * k, 128), :], pe_v)

        def fire_idx(r, b, sb=sb):
            pltpu.async_copy(
                tok_hbm.at[pl.ds((wid * ROWS_PER_W + r) * SEQ_BLOCKS + sb, 8), :],
                idx[b], sem_i[b])

        def drain_idx(b):
            pltpu.make_async_copy(
                tok_hbm.at[pl.ds(0, 8), :], idx[b], sem_i[b]).wait()

        def fire_gather(b, kb=kb):
            pltpu.async_copy(table_hbm.at[idx[b].at[kb]], rows[b], sem_g[b])

        def drain_gather(b):
            pltpu.make_async_copy(
                table_hbm.at[pl.ds(0, 128), :], rows[b], sem_g[b]).wait()

        def add_pe(b, is_tail=is_tail):
            dst = tailv[b] if is_tail else outv[b]
            unroll = 4 if is_tail else 8
            n = TAIL if is_tail else 128

            def add_body(i, c):
                for u in range(unroll):
                    p = i * unroll + u
                    for v in range(DIM // 16):
                        sl = pl.ds(v * 16, 16)
                        dst[p, sl] = rows[b][p, sl] + pe_v[p, sl]
                return c
            lax.fori_loop(0, n // unroll, add_body, 0)

        def fire_store(r, b, k=k, is_tail=is_tail):
            g = wid * ROWS_PER_W + r
            if not is_tail:
                pltpu.async_copy(
                    outv[b], out_hbm.at[g, pl.ds(128 * k, 128), :], sem_s[b])
            else:
                pltpu.async_copy(
                    tailv[b].at[pl.ds(0, 88), :],
                    out_hbm.at[g, pl.ds(1408, 88), :], sem_s[b])
                pltpu.async_copy(
                    tailv[b].at[pl.ds(88, 4), :],
                    out_hbm.at[g, pl.ds(1496, 4), :], sem_s[b])

        def drain_store(b, is_tail=is_tail):
            if not is_tail:
                pltpu.make_async_copy(
                    outv[b], out_hbm.at[0, pl.ds(0, 128), :], sem_s[b]).wait()
            else:
                pltpu.make_async_copy(
                    tailv[b].at[pl.ds(0, 88), :],
                    out_hbm.at[0, pl.ds(1408, 88), :], sem_s[b]).wait()
                pltpu.make_async_copy(
                    tailv[b].at[pl.ds(88, 4), :],
                    out_hbm.at[0, pl.ds(1496, 4), :], sem_s[b]).wait()

        fire_idx(0, 0)

        def slot_body(i, carry):
            for u in range(NBUF):
                r = i * NBUF + u
                pb = (u + NBUF - 1) % NBUF

                @pl.when(r >= 1)
                def _():
                    drain_gather(pb)
                    add_pe(pb)
                    fire_store(r - 1, pb)

                @pl.when(r >= NBUF)
                def _():
                    drain_store(u)

                drain_idx(u)
                fire_gather(u)

                @pl.when(r < ROWS_PER_W - 1)
                def _():
                    fire_idx(r + 1, (u + 1) % NBUF)
            return carry

        lax.fori_loop(0, ROWS_PER_W // NBUF, slot_body, 0)

        last_b = (ROWS_PER_W - 1) % NBUF
        drain_gather(last_b)
        add_pe(last_b)
        fire_store(ROWS_PER_W - 1, last_b)
        for b in range(NBUF):
            drain_store(b)


def kernel(tokens, table):
    pe = _pe_table()
    table128 = jnp.pad(table, ((0, 0), (0, 128 - DIM)))
    tok_blocks = jnp.pad(tokens, ((0, 0), (0, SEQ_PAD - SEQ))).reshape(-1, 128)
    return _seq_encode(tok_blocks, pe, table128)


# speedup vs baseline: 1.1120x; 1.1120x over previous
import functools
import math

import jax
import jax.numpy as jnp
from jax import lax
from jax.experimental import pallas as pl
from jax.experimental.pallas import tpu as pltpu
from jax.experimental.pallas import tpu_sc as plsc

VOCAB = 100000
DIM = 64
SEQ = 1500
BATCH = 1024
SEQ_PAD = 2048
SEQ_BLOCKS = SEQ_PAD // 128
N_CHUNK = 12
TAIL = SEQ - 11 * 128

NC = 2
NS = 16
NW = NC * NS
ROWS_PER_W = BATCH // NW

NG = 4
LOOKAHEAD = 3


def _pe_table():
    position = jnp.arange(SEQ, dtype=jnp.float32)[:, None]
    div_term = jnp.exp(
        jnp.arange(0, DIM, 2, dtype=jnp.float32) * (-(math.log(10000.0) / DIM))
    )
    ang = position * div_term
    pe = jnp.zeros((SEQ, DIM), dtype=jnp.float32)
    pe = pe.at[:, 0::2].set(jnp.sin(ang))
    pe = pe.at[:, 1::2].set(jnp.cos(ang))
    return jnp.pad(pe, ((0, N_CHUNK * 128 - SEQ), (0, 0)))


@functools.partial(
    pl.kernel,
    mesh=plsc.VectorSubcoreMesh(core_axis_name="c", subcore_axis_name="s"),
    out_type=jax.ShapeDtypeStruct((BATCH, SEQ, DIM), jnp.float32),
    scratch_types=(
        [pltpu.VMEM((128, DIM), jnp.float32)]
        + [pltpu.VMEM((8, 128), jnp.int32) for _ in range(NG)]
        + [pltpu.VMEM((128, 128), jnp.float32) for _ in range(NG)]
        + [pltpu.VMEM((128, DIM), jnp.float32)]
        + [pltpu.VMEM((TAIL, DIM), jnp.float32)]
        + [pltpu.SemaphoreType.DMA for _ in range(2 * NG + 1)]
    ),
)
def _seq_encode(tok_hbm, pe_hbm, table_hbm, out_hbm, pe_v, *scratch):
    idx = scratch[:NG]
    rows = scratch[NG:2 * NG]
    outv = scratch[2 * NG]
    tailv = scratch[2 * NG + 1]
    sem_i = scratch[2 * NG + 2:3 * NG + 2]
    sem_g = scratch[3 * NG + 2:4 * NG + 2]
    sem_s = scratch[4 * NG + 2]
    wid = lax.axis_index("s") * NC + lax.axis_index("c")

    def run_chunk(kk, is_tail):
        static = isinstance(kk, int)
        off = kk * 128 if static else pl.multiple_of(kk * 128, 8)
        sblk = (kk // 8) * 8
        kb = kk % 8

        pltpu.sync_copy(pe_hbm.at[pl.ds(off, 128), :], pe_v)

        def fire_idx(r, b):
            brow = (wid * ROWS_PER_W + r) * SEQ_BLOCKS + sblk
            if not static:
                brow = pl.multiple_of(brow, 8)
            pltpu.async_copy(tok_hbm.at[pl.ds(brow, 8), :], idx[b], sem_i[b])

        def drain_idx(b):
            pltpu.make_async_copy(
                tok_hbm.at[pl.ds(0, 8), :], idx[b], sem_i[b]).wait()

        def fire_gather(b):
            pltpu.async_copy(table_hbm.at[idx[b].at[kb]], rows[b], sem_g[b])

        def drain_gather(b):
            pltpu.make_async_copy(
                table_hbm.at[pl.ds(0, 128), :], rows[b], sem_g[b]).wait()

        def add_pe(gb):
            dst = tailv if is_tail else outv
            unroll = 4 if is_tail else 8
            n = TAIL if is_tail else 128

            def add_body(i, c):
                for u in range(unroll):
                    p = i * unroll + u
                    for v in range(DIM // 16):
                        sl = pl.ds(v * 16, 16)
                        dst[p, sl] = rows[gb][p, sl] + pe_v[p, sl]
                return c
            lax.fori_loop(0, n // unroll, add_body, 0)

        def fire_store(r):
            g = wid * ROWS_PER_W + r
            if not is_tail:
                pltpu.async_copy(
                    outv, out_hbm.at[g, pl.ds(off, 128), :], sem_s)
            else:
                pltpu.async_copy(
                    tailv.at[pl.ds(0, 88), :],
                    out_hbm.at[g, pl.ds(1408, 88), :], sem_s)
                pltpu.async_copy(
                    tailv.at[pl.ds(88, 4), :],
                    out_hbm.at[g, pl.ds(1496, 4), :], sem_s)

        def drain_store():
            if not is_tail:
                pltpu.make_async_copy(
                    outv, out_hbm.at[0, pl.ds(0, 128), :], sem_s).wait()
            else:
                pltpu.make_async_copy(
                    tailv.at[pl.ds(0, 88), :],
                    out_hbm.at[0, pl.ds(1408, 88), :], sem_s).wait()
                pltpu.make_async_copy(
                    tailv.at[pl.ds(88, 4), :],
                    out_hbm.at[0, pl.ds(1496, 4), :], sem_s).wait()

        for r0 in range(NG):
            fire_idx(r0, r0)
        for r0 in range(LOOKAHEAD):
            drain_idx(r0)
            fire_gather(r0)

        def slot_body(i, carry):
            for u in range(NG):
                r = i * NG + u

                drain_gather(u)

                @pl.when(r >= 1)
                def _():
                    drain_store()

                add_pe(u)
                fire_store(r)

                @pl.when(r + NG < ROWS_PER_W)
                def _():
                    fire_idx(r + NG, u)

                @pl.when(r + LOOKAHEAD < ROWS_PER_W)
                def _():
                    drain_idx((u + LOOKAHEAD) % NG)
                    fire_gather((u + LOOKAHEAD) % NG)
            return carry

        lax.fori_loop(0, ROWS_PER_W // NG, slot_body, 0)
        drain_store()

    def chunk_body(kk, carry):
        run_chunk(kk, False)
        return carry

    lax.fori_loop(0, N_CHUNK - 1, chunk_body, 0)
    run_chunk(N_CHUNK - 1, True)


def kernel(tokens, table):
    pe = _pe_table()
    table128 = jnp.pad(table, ((0, 0), (0, 128 - DIM)))
    tok_blocks = jnp.pad(tokens, ((0, 0), (0, SEQ_PAD - SEQ))).reshape(-1, 128)
    return _seq_encode(tok_blocks, pe, table128)
